# baseline (speedup 1.0000x reference)
import numpy as np
import jax
import jax.numpy as jnp
from jax.experimental import pallas as pl
from jax.experimental.pallas import tpu as pltpu

N = 50000
E = 800000
NUM_LABELS = 1000
NUM_TYPES = 100
EMB = 16
HEADS = 4
HEAD_DIM = 16
GAT_HID = HEADS * HEAD_DIM
LSTM_HID = 128
OUT_DIM = 4
NUM_GRAPHS = 64
D_IN = 2 * EMB + 1


def _pe_row0(d_model):
    pe = np.zeros((1, d_model), dtype=np.float32)
    pos = np.arange(0, 1, dtype=np.float32)[:, None]
    div = np.exp(np.arange(0, d_model, 2, dtype=np.float32) * (-np.log(10000.0) / d_model))
    pe[:, 0::2] = np.sin(pos * div)
    if d_model % 2 == 0:
        pe[:, 1::2] = np.cos(pos * div)
    return jnp.asarray(pe)


def _edge_softmax(logits, dst, n):
    m = jax.ops.segment_max(logits, dst, num_segments=n)
    m = jnp.where(jnp.isfinite(m), m, 0.0)
    e = jnp.exp(logits - m[dst])
    s = jax.ops.segment_sum(e, dst, num_segments=n)
    return e / (s[dst] + 1e-16)


def _gatv2(x, src, dst, Wl, bl, Wr, br, att, bias):
    n = x.shape[0]
    xl = (x @ Wl.T + bl).reshape(n, HEADS, HEAD_DIM)
    xr = (x @ Wr.T + br).reshape(n, HEADS, HEAD_DIM)
    xj = xl[src]
    xi = xr[dst]
    e = jax.nn.leaky_relu(xi + xj, 0.2)
    logits = jnp.einsum('ehd,hd->eh', e, att)
    alpha = _edge_softmax(logits, dst, n)
    out = jax.ops.segment_sum(xj * alpha[:, :, None], dst, num_segments=n)
    return out.reshape(n, GAT_HID) + bias


def _graph_norm(x, w, b, ms):
    mean = jnp.mean(x, axis=0, keepdims=True)
    out = x - mean * ms
    var = jnp.mean(out * out, axis=0, keepdims=True)
    return out / jnp.sqrt(var + 1e-5) * w + b


def _lstm_cell(x, Wih, Whh, bih, bhh):
    h0 = jnp.zeros((x.shape[0], LSTM_HID), dtype=x.dtype)
    g = x @ Wih.T + h0 @ Whh.T + bih + bhh
    i, f, gg, o = jnp.split(g, 4, axis=1)
    i = jax.nn.sigmoid(i)
    f = jax.nn.sigmoid(f)
    gg = jnp.tanh(gg)
    o = jax.nn.sigmoid(o)
    c = i * gg
    return o * jnp.tanh(c)


def _layer_norm(x, w, b):
    mu = jnp.mean(x, axis=-1, keepdims=True)
    var = jnp.var(x, axis=-1, keepdims=True)
    return (x - mu) / jnp.sqrt(var + 1e-5) * w + b


def _head_kernel(pooled_ref, wfc_ref, bfc_ref, out_ref):
    logits = jnp.dot(pooled_ref[...], wfc_ref[...].T,
                     preferred_element_type=jnp.float32) + bfc_ref[...]
    m = jnp.max(logits, axis=1, keepdims=True)
    s = jnp.log(jnp.sum(jnp.exp(logits - m), axis=1, keepdims=True))
    out_ref[...] = logits - m - s


def _head(pooled, Wfc, bfc):
    return pl.pallas_call(
        _head_kernel,
        out_shape=jax.ShapeDtypeStruct((NUM_GRAPHS, OUT_DIM), jnp.float32),
    )(pooled, Wfc, bfc.reshape(1, OUT_DIM))


def kernel(node_labels, node_types, node_scalar, edge_index, batch, label_table, type_table, Wp, bp, Wl1, bl1, Wr1, br1, att1, bias1, gn1_w, gn1_b, gn1_ms, Wl2, bl2, Wr2, br2, att2, bias2, gn2_w, gn2_b, gn2_ms, Wres, bres, Wih1, Whh1, bih1, bhh1, Wih2, Whh2, bih2, bhh2, ln_w, ln_b, Wfc, bfc):
    loops = jnp.arange(N, dtype=edge_index.dtype)
    src = jnp.concatenate([edge_index[0], loops])
    dst = jnp.concatenate([edge_index[1], loops])
    x = jnp.concatenate([label_table[node_labels], type_table[node_types], node_scalar], axis=1)
    x = x + _pe_row0(D_IN)
    xp = x @ Wp.T + bp
    x1 = jax.nn.elu(_graph_norm(_gatv2(xp, src, dst, Wl1, bl1, Wr1, br1, att1, bias1), gn1_w, gn1_b, gn1_ms))
    x2 = jax.nn.elu(_graph_norm(_gatv2(x1, src, dst, Wl2, bl2, Wr2, br2, att2, bias2), gn2_w, gn2_b, gn2_ms) + x1 @ Wres.T + bres)
    h1 = _lstm_cell(x2, Wih1, Whh1, bih1, bhh1)
    h2 = _lstm_cell(h1, Wih2, Whh2, bih2, bhh2)
    xn = _layer_norm(h2, ln_w, ln_b)
    counts = jax.ops.segment_sum(jnp.ones((N,), dtype=xn.dtype), batch, num_segments=NUM_GRAPHS)
    pooled = jax.ops.segment_sum(xn, batch, num_segments=NUM_GRAPHS) / jnp.maximum(counts, 1.0)[:, None]
    return _head(pooled, Wfc, bfc)


# speedup vs baseline: 21.5383x; 21.5383x over previous
import functools
import numpy as np
import jax
import jax.numpy as jnp
from jax import lax
from jax.experimental import pallas as pl
from jax.experimental.pallas import tpu as pltpu
from jax.experimental.pallas import tpu_sc as plsc

N = 50000
E = 800000
NUM_LABELS = 1000
NUM_TYPES = 100
EMB = 16
HEADS = 4
HEAD_DIM = 16
GAT_HID = HEADS * HEAD_DIM
LSTM_HID = 128
OUT_DIM = 4
NUM_GRAPHS = 64

NC = 2
NS = 16
L = 16
W128 = 128

QR = 12500
QP = 12544
NQ = 4
NP = NQ * QP
DUMMY_Q = 12520
Q_ROWS_PER_TILE = QP // NS

E_TOT = E + N
CHUNK = 128
TILE_E = 53248
N_CHUNK = TILE_E // CHUNK
ET_PAD = NS * TILE_E
MSG_TILE_E = ET_PAD // (NC * NS)
MSG_CHUNKS = MSG_TILE_E // CHUNK

NB = 16
BN = NP // NB

LAB_PAD = 1024
TYP_PAD = 128
DIN_PAD = 64


_MESH = plsc.VectorSubcoreMesh(core_axis_name="c", subcore_axis_name="s")
_SC_PARAMS = pltpu.CompilerParams(needs_layout_passes=False)


def _msg_body(srcp_hbm, dstp_hbm, xl_hbm, xr_hbm, att_hbm, msg_hbm,
              idx_src, idx_dst, xl_rows, xr_rows, msg_buf, att_v,
              sem0, sem1):
    cid = lax.axis_index("c")
    sid = lax.axis_index("s")
    wid = sid * NC + cid

    pltpu.sync_copy(att_hbm, att_v)
    lane = lax.iota(jnp.int32, L)

    @pl.loop(0, CHUNK)
    def _(i):
        for j in range(W128 // L):
            msg_buf[i, pl.ds(j * L, L)] = jnp.zeros((L,), jnp.float32)

    e_base0 = wid * MSG_TILE_E

    @pl.loop(0, MSG_CHUNKS)
    def _(g):
        e_base = e_base0 + g * CHUNK
        pltpu.sync_copy(srcp_hbm.at[pl.ds(e_base, CHUNK)], idx_src)
        pltpu.sync_copy(dstp_hbm.at[pl.ds(e_base, CHUNK)], idx_dst)
        cp0 = pltpu.async_copy(xl_hbm.at[idx_src], xl_rows, sem0)
        cp1 = pltpu.async_copy(xr_hbm.at[idx_dst], xr_rows, sem1)
        cp0.wait()
        cp1.wait()

        @pl.loop(0, CHUNK)
        def _(i):
            lrow = jnp.full((L,), -60.0, jnp.float32)
            for h in range(HEADS):
                xj = xl_rows[i, pl.ds(h * L, L)]
                xi = xr_rows[i, pl.ds(h * L, L)]
                s = xi + xj
                e = jnp.maximum(s, s * 0.2)
                logit = jnp.sum(e * att_v[h])
                lv = jnp.broadcast_to(logit, (L,))
                msg_buf[i, pl.ds(h * L, L)] = xj * jnp.exp(lv)
                lrow = jnp.where(lane == h, lv, lrow)
            msg_buf[i, pl.ds(GAT_HID, L)] = jnp.exp(lrow)

        pltpu.sync_copy(msg_buf, msg_hbm.at[pl.ds(e_base, CHUNK)])


def _msg_phase(srcp, dstp, xl, xr, att):
    k = pl.kernel(
        _msg_body,
        out_type=jax.ShapeDtypeStruct((ET_PAD, W128), jnp.float32),
        mesh=_MESH,
        compiler_params=_SC_PARAMS,
        scratch_types=[
            pltpu.VMEM((CHUNK,), jnp.int32),
            pltpu.VMEM((CHUNK,), jnp.int32),
            pltpu.VMEM((CHUNK, W128), jnp.float32),
            pltpu.VMEM((CHUNK, W128), jnp.float32),
            pltpu.VMEM((CHUNK, W128), jnp.float32),
            pltpu.VMEM((HEADS, L), jnp.float32),
            pltpu.SemaphoreType.DMA,
            pltpu.SemaphoreType.DMA,
        ],
    )
    return k(srcp, dstp, xl, xr, att)


def _scatter_body(dstp_hbm, msg_hbm, out_hbm,
                  idx_dst, idx_adj, rows, acc, sem0):
    cid = lax.axis_index("c")
    sid = lax.axis_index("s")
    tile_row0 = sid * Q_ROWS_PER_TILE
    e_base0 = sid * TILE_E

    for r in range(2):
        q = 2 * cid + r
        q_base = q * QP

        @pl.loop(0, CHUNK)
        def _(i):
            for j in range(W128 // L):
                rows[i, pl.ds(j * L, L)] = jnp.zeros((L,), jnp.float32)

        nfull = Q_ROWS_PER_TILE // CHUNK
        for k in range(nfull):
            pltpu.sync_copy(rows, acc.at[pl.ds(tile_row0 + k * CHUNK, CHUNK)])
        rem = Q_ROWS_PER_TILE - nfull * CHUNK
        if rem:
            pltpu.sync_copy(rows.at[pl.ds(0, rem)],
                            acc.at[pl.ds(tile_row0 + nfull * CHUNK, rem)])
        plsc.subcore_barrier()

        @pl.loop(0, N_CHUNK)
        def _(g):
            e_base = e_base0 + g * CHUNK
            pltpu.sync_copy(dstp_hbm.at[pl.ds(e_base, CHUNK)], idx_dst)
            cp0 = pltpu.async_copy(msg_hbm.at[pl.ds(e_base, CHUNK)], rows, sem0)

            @pl.loop(0, CHUNK // L)
            def _(j):
                d = idx_dst[pl.ds(j * L, L)]
                local = d - q_base
                ok = (local >= 0) & (local < QP)
                idx_adj[pl.ds(j * L, L)] = jnp.where(ok, local, DUMMY_Q)

            cp0.wait()
            pltpu.sync_copy(rows, acc.at[idx_adj], add=True)

        plsc.subcore_barrier()
        pltpu.sync_copy(acc.at[pl.ds(tile_row0, Q_ROWS_PER_TILE)],
                        out_hbm.at[pl.ds(q_base + tile_row0, Q_ROWS_PER_TILE)])
        plsc.subcore_barrier()


def _scatter_phase(dstp, msg):
    k = pl.kernel(
        _scatter_body,
        out_type=jax.ShapeDtypeStruct((NP, W128), jnp.float32),
        mesh=_MESH,
        compiler_params=_SC_PARAMS,
        scratch_types=[
            pltpu.VMEM((CHUNK,), jnp.int32),
            pltpu.VMEM((CHUNK,), jnp.int32),
            pltpu.VMEM((CHUNK, W128), jnp.float32),
            pltpu.VMEM_SHARED((QP, W128), jnp.float32),
            pltpu.SemaphoreType.DMA,
        ],
    )
    return k(dstp, msg)


def _edge_phase(srcp, dstp, xl, xr, att):
    msg = _msg_phase(srcp, dstp, xl, xr, att)
    return _scatter_phase(dstp, msg)


def _k1_body(lab_ref, typ_ref, scal_ref, labt_ref, typt_ref,
             wp_ref, bp_ref, wl_ref, bl_ref, wr_ref, br_ref,
             xl_ref, xr_ref):
    lab = lab_ref[...]
    typ = typ_ref[...]
    iota_l = lax.broadcasted_iota(jnp.int32, (BN, LAB_PAD), 1)
    iota_t = lax.broadcasted_iota(jnp.int32, (BN, TYP_PAD), 1)
    oh_l = (lab == iota_l).astype(jnp.float32)
    oh_t = (typ == iota_t).astype(jnp.float32)
    emb_l = jnp.dot(oh_l, labt_ref[...], preferred_element_type=jnp.float32)
    emb_t = jnp.dot(oh_t, typt_ref[...], preferred_element_type=jnp.float32)
    x = jnp.concatenate(
        [emb_l, emb_t, scal_ref[...],
         jnp.zeros((BN, DIN_PAD - 2 * EMB - 1), jnp.float32)], axis=1)
    xp = jnp.dot(x, wp_ref[...], preferred_element_type=jnp.float32) + bp_ref[...]
    xl_ref[...] = jnp.dot(xp, wl_ref[...],
                          preferred_element_type=jnp.float32) + bl_ref[...]
    xr_ref[...] = jnp.dot(xp, wr_ref[...],
                          preferred_element_type=jnp.float32) + br_ref[...]


def _k1(lab, typ, scal, labt, typt, wpT, bp, wlT, bl, wrT, br):
    full = lambda s: pl.BlockSpec(s, lambda i: (0, 0))
    return pl.pallas_call(
        _k1_body,
        grid=(NB,),
        in_specs=[
            pl.BlockSpec((BN, 1), lambda i: (i, 0)),
            pl.BlockSpec((BN, 1), lambda i: (i, 0)),
            pl.BlockSpec((BN, 1), lambda i: (i, 0)),
            full((LAB_PAD, EMB)),
            full((TYP_PAD, EMB)),
            full((DIN_PAD, GAT_HID)),
            full((1, GAT_HID)),
            full((GAT_HID, W128)),
            full((1, W128)),
            full((GAT_HID, W128)),
            full((1, W128)),
        ],
        out_specs=[
            pl.BlockSpec((BN, W128), lambda i: (i, 0)),
            pl.BlockSpec((BN, W128), lambda i: (i, 0)),
        ],
        out_shape=[
            jax.ShapeDtypeStruct((NP, W128), jnp.float32),
            jax.ShapeDtypeStruct((NP, W128), jnp.float32),
        ],
    )(lab, typ, scal, labt, typt, wpT, bp, wlT, bl, wrT, br)


def _gat_from_acc(acc, bias_row):
    numer = acc[:, :GAT_HID]
    den4 = acc[:, GAT_HID:GAT_HID + HEADS]
    dparts = [jnp.broadcast_to(den4[:, h][:, None], (acc.shape[0], L))
              for h in range(HEADS)]
    den = jnp.concatenate(dparts, axis=1)
    return numer / (den + 1e-16) + bias_row


def _valid_mask(i):
    r = i * BN + lax.broadcasted_iota(jnp.int32, (BN, 1), 0)
    ok = (r - (r // QP) * QP) < QR
    return ok.astype(jnp.float32)


def _sums_body(acc_ref, bias_ref, out_ref):
    i = pl.program_id(0)
    gat = _gat_from_acc(acc_ref[...], bias_ref[...])
    m = _valid_mask(i)
    g = gat * m
    s1 = jnp.sum(g, axis=0, keepdims=True)
    s2 = jnp.sum(g * gat, axis=0, keepdims=True)
    part = jnp.concatenate(
        [s1, s2, jnp.zeros((6, GAT_HID), jnp.float32)], axis=0)

    @pl.when(i == 0)
    def _():
        out_ref[...] = jnp.zeros_like(out_ref)

    out_ref[...] += part


def _sums(acc, bias_row):
    return pl.pallas_call(
        _sums_body,
        grid=(NB,),
        in_specs=[
            pl.BlockSpec((BN, W128), lambda i: (i, 0)),
            pl.BlockSpec((1, GAT_HID), lambda i: (0, 0)),
        ],
        out_specs=pl.BlockSpec((8, GAT_HID), lambda i: (0, 0)),
        out_shape=jax.ShapeDtypeStruct((8, GAT_HID), jnp.float32),
    )(acc, bias_row)


def _graph_norm_cols(gat, sums_ref, gw, gb, gms):
    s1 = sums_ref[0, :][None, :]
    s2 = sums_ref[1, :][None, :]
    mean = s1 / float(N)
    ex2 = s2 / float(N)
    var = ex2 - (2.0 * gms - gms * gms) * mean * mean
    out = gat - mean * gms
    return out * lax.rsqrt(var + 1e-5) * gw + gb


def _k2b_body(acc_ref, sums_ref, bias_ref, gw_ref, gb_ref, gms_ref,
              wl_ref, bl_ref, wr_ref, br_ref, wres_ref, bres_ref,
              xl_ref, xr_ref, res_ref):
    gat = _gat_from_acc(acc_ref[...], bias_ref[...])
    x1 = _graph_norm_cols(gat, sums_ref, gw_ref[...], gb_ref[...], gms_ref[...])
    x1 = jnp.where(x1 > 0, x1, jnp.exp(x1) - 1.0)
    xl_ref[...] = jnp.dot(x1, wl_ref[...],
                          preferred_element_type=jnp.float32) + bl_ref[...]
    xr_ref[...] = jnp.dot(x1, wr_ref[...],
                          preferred_element_type=jnp.float32) + br_ref[...]
    res_ref[...] = jnp.dot(x1, wres_ref[...],
                           preferred_element_type=jnp.float32) + bres_ref[...]


def _k2b(acc, sums, bias_row, gw, gb, gms, wlT, bl, wrT, br, wresT, bres):
    full = lambda s: pl.BlockSpec(s, lambda i: (0, 0))
    return pl.pallas_call(
        _k2b_body,
        grid=(NB,),
        in_specs=[
            pl.BlockSpec((BN, W128), lambda i: (i, 0)),
            full((8, GAT_HID)),
            full((1, GAT_HID)),
            full((1, GAT_HID)),
            full((1, GAT_HID)),
            full((1, GAT_HID)),
            full((GAT_HID, W128)),
            full((1, W128)),
            full((GAT_HID, W128)),
            full((1, W128)),
            full((GAT_HID, GAT_HID)),
            full((1, GAT_HID)),
        ],
        out_specs=[
            pl.BlockSpec((BN, W128), lambda i: (i, 0)),
            pl.BlockSpec((BN, W128), lambda i: (i, 0)),
            pl.BlockSpec((BN, GAT_HID), lambda i: (i, 0)),
        ],
        out_shape=[
            jax.ShapeDtypeStruct((NP, W128), jnp.float32),
            jax.ShapeDtypeStruct((NP, W128), jnp.float32),
            jax.ShapeDtypeStruct((NP, GAT_HID), jnp.float32),
        ],
    )(acc, sums, bias_row, gw, gb, gms, wlT, bl, wrT, br, wresT, bres)


def _k3b_body(acc_ref, sums_ref, bias_ref, gw_ref, gb_ref, gms_ref,
              res_ref, batch_ref, wih1_ref, b1_ref, wih2_ref, b2_ref,
              lnw_ref, lnb_ref, pooled_ref, counts_ref):
    i = pl.program_id(0)
    gat = _gat_from_acc(acc_ref[...], bias_ref[...])
    x2 = _graph_norm_cols(gat, sums_ref, gw_ref[...], gb_ref[...],
                          gms_ref[...]) + res_ref[...]
    x2 = jnp.where(x2 > 0, x2, jnp.exp(x2) - 1.0)

    def cell(x, wT, brow):
        g = jnp.dot(x, wT, preferred_element_type=jnp.float32) + brow
        gi = jax.nn.sigmoid(g[:, :LSTM_HID])
        gg = jnp.tanh(g[:, 2 * LSTM_HID:3 * LSTM_HID])
        go = jax.nn.sigmoid(g[:, 3 * LSTM_HID:])
        return go * jnp.tanh(gi * gg)

    h1 = cell(x2, wih1_ref[...], b1_ref[...])
    h2 = cell(h1, wih2_ref[...], b2_ref[...])

    mu = jnp.mean(h2, axis=1, keepdims=True)
    var = jnp.mean((h2 - mu) * (h2 - mu), axis=1, keepdims=True)
    xn = (h2 - mu) * lax.rsqrt(var + 1e-5) * lnw_ref[...] + lnb_ref[...]

    b = batch_ref[...]
    iota_g = lax.broadcasted_iota(jnp.int32, (BN, NUM_GRAPHS), 1)
    oh = (b == iota_g).astype(jnp.float32)
    pooled_part = lax.dot_general(oh, xn, (((0,), (0,)), ((), ())),
                                  preferred_element_type=jnp.float32)
    counts_part = lax.dot_general(oh, jnp.ones((BN, LSTM_HID), jnp.float32),
                                  (((0,), (0,)), ((), ())),
                                  preferred_element_type=jnp.float32)

    @pl.when(i == 0)
    def _():
        pooled_ref[...] = jnp.zeros_like(pooled_ref)
        counts_ref[...] = jnp.zeros_like(counts_ref)

    pooled_ref[...] += pooled_part
    counts_ref[...] += counts_part


def _k3b(acc, sums, bias_row, gw, gb, gms, res, batch_col,
         wih1T, b1, wih2T, b2, lnw, lnb):
    full = lambda s: pl.BlockSpec(s, lambda i: (0, 0))
    return pl.pallas_call(
        _k3b_body,
        grid=(NB,),
        in_specs=[
            pl.BlockSpec((BN, W128), lambda i: (i, 0)),
            full((8, GAT_HID)),
            full((1, GAT_HID)),
            full((1, GAT_HID)),
            full((1, GAT_HID)),
            full((1, GAT_HID)),
            pl.BlockSpec((BN, GAT_HID), lambda i: (i, 0)),
            pl.BlockSpec((BN, 1), lambda i: (i, 0)),
            full((GAT_HID, 4 * LSTM_HID)),
            full((1, 4 * LSTM_HID)),
            full((LSTM_HID, 4 * LSTM_HID)),
            full((1, 4 * LSTM_HID)),
            full((1, LSTM_HID)),
            full((1, LSTM_HID)),
        ],
        out_specs=[
            pl.BlockSpec((NUM_GRAPHS, LSTM_HID), lambda i: (0, 0)),
            pl.BlockSpec((NUM_GRAPHS, LSTM_HID), lambda i: (0, 0)),
        ],
        out_shape=[
            jax.ShapeDtypeStruct((NUM_GRAPHS, LSTM_HID), jnp.float32),
            jax.ShapeDtypeStruct((NUM_GRAPHS, LSTM_HID), jnp.float32),
        ],
    )(acc, sums, bias_row, gw, gb, gms, res, batch_col,
      wih1T, b1, wih2T, b2, lnw, lnb)


def _head_body(pooled_ref, counts_ref, wfc_ref, bfc_ref, out_ref):
    pooled = pooled_ref[...] / jnp.maximum(counts_ref[...], 1.0)
    logits = jnp.dot(pooled, wfc_ref[...],
                     preferred_element_type=jnp.float32) + bfc_ref[...]
    m = jnp.max(logits, axis=1, keepdims=True)
    s = jnp.log(jnp.sum(jnp.exp(logits - m), axis=1, keepdims=True))
    out_ref[...] = logits - m - s


def _head(pooled, counts, wfcT, bfc):
    return pl.pallas_call(
        _head_body,
        out_shape=jax.ShapeDtypeStruct((NUM_GRAPHS, OUT_DIM), jnp.float32),
    )(pooled, counts, wfcT, bfc)


def _pad_nodes(a, fill=0):
    pad = jnp.full((QP - QR,) + a.shape[1:], fill, a.dtype)
    parts = []
    for q in range(NQ):
        parts.append(a[q * QR:(q + 1) * QR])
        parts.append(pad)
    return jnp.concatenate(parts, axis=0)


def kernel(node_labels, node_types, node_scalar, edge_index, batch, label_table, type_table, Wp, bp, Wl1, bl1, Wr1, br1, att1, bias1, gn1_w, gn1_b, gn1_ms, Wl2, bl2, Wr2, br2, att2, bias2, gn2_w, gn2_b, gn2_ms, Wres, bres, Wih1, Whh1, bih1, bhh1, Wih2, Whh2, bih2, bhh2, ln_w, ln_b, Wfc, bfc):
    del Whh1, Whh2

    loops = jnp.arange(N, dtype=jnp.int32)
    src = jnp.concatenate([edge_index[0].astype(jnp.int32), loops])
    dst = jnp.concatenate([edge_index[1].astype(jnp.int32), loops])
    srcp = src + (QP - QR) * (src // QR)
    dstp = dst + (QP - QR) * (dst // QR)
    srcp = jnp.concatenate([srcp, jnp.zeros((ET_PAD - E_TOT,), jnp.int32)])
    dstp = jnp.concatenate([dstp, jnp.full((ET_PAD - E_TOT,), DUMMY_Q, jnp.int32)])

    lab_p = _pad_nodes(node_labels.astype(jnp.int32)[:, None])
    typ_p = _pad_nodes(node_types.astype(jnp.int32)[:, None])
    scal_p = _pad_nodes(node_scalar.astype(jnp.float32))
    batch_p = _pad_nodes(batch.astype(jnp.int32)[:, None], fill=-1)

    labt = jnp.zeros((LAB_PAD, EMB), jnp.float32).at[:NUM_LABELS].set(label_table)
    typt = jnp.zeros((TYP_PAD, EMB), jnp.float32).at[:NUM_TYPES].set(type_table)
    wpT = jnp.zeros((DIN_PAD, GAT_HID), jnp.float32).at[:2 * EMB + 1].set(Wp.T)
    row = lambda v: v[None, :]
    wide = lambda w: jnp.zeros((GAT_HID, W128), jnp.float32).at[:, :GAT_HID].set(w.T)
    widb = lambda b: jnp.zeros((1, W128), jnp.float32).at[:, :GAT_HID].set(b[None, :])

    xl1, xr1 = _k1(lab_p, typ_p, scal_p, labt, typt,
                   wpT, row(bp), wide(Wl1), widb(bl1), wide(Wr1), widb(br1))

    acc1 = _edge_phase(srcp, dstp, xl1, xr1, att1)
    sums1 = _sums(acc1, row(bias1))
    xl2, xr2, res = _k2b(acc1, sums1, row(bias1), row(gn1_w), row(gn1_b),
                         row(gn1_ms), wide(Wl2), widb(bl2), wide(Wr2),
                         widb(br2), Wres.T, row(bres))

    acc2 = _edge_phase(srcp, dstp, xl2, xr2, att2)
    sums2 = _sums(acc2, row(bias2))
    pooled, counts = _k3b(acc2, sums2, row(bias2), row(gn2_w), row(gn2_b),
                          row(gn2_ms), res, batch_p,
                          Wih1.T, row(bih1 + bhh1), Wih2.T, row(bih2 + bhh2),
                          row(ln_w), row(ln_b))

    return _head(pooled, counts, Wfc.T, row(bfc))


# speedup vs baseline: 24.1991x; 1.1235x over previous
import functools
import numpy as np
import jax
import jax.numpy as jnp
from jax import lax
from jax.experimental import pallas as pl
from jax.experimental.pallas import tpu as pltpu
from jax.experimental.pallas import tpu_sc as plsc

N = 50000
E = 800000
NUM_LABELS = 1000
NUM_TYPES = 100
EMB = 16
HEADS = 4
HEAD_DIM = 16
GAT_HID = HEADS * HEAD_DIM
LSTM_HID = 128
OUT_DIM = 4
NUM_GRAPHS = 64

NC = 2
NS = 16
L = 16
W128 = 128

QR = 12500
QP = 12544
NQ = 4
NP = NQ * QP
DUMMY_Q = 12520
Q_ROWS_PER_TILE = QP // NS

E_TOT = E + N
CHUNK = 128
TILE_E = 53248
N_CHUNK = TILE_E // CHUNK
ET_PAD = NS * TILE_E
MSG_TILE_E = ET_PAD // (NC * NS)
MSG_CHUNKS = MSG_TILE_E // CHUNK

NB = 16
BN = NP // NB

LAB_PAD = 1024
TYP_PAD = 128
DIN_PAD = 64


_MESH = plsc.VectorSubcoreMesh(core_axis_name="c", subcore_axis_name="s")
_SC_PARAMS = pltpu.CompilerParams(needs_layout_passes=False)


def _msg_body(srcp_hbm, dstp_hbm, xl_hbm, xr_hbm, att_hbm, msg_hbm,
              idx_src0, idx_dst0, xl_rows0, xr_rows0, msg_buf0,
              idx_src1, idx_dst1, xl_rows1, xr_rows1, msg_buf1, att_v,
              semA0, semA1, semB0, semB1, semW0, semW1):
    cid = lax.axis_index("c")
    sid = lax.axis_index("s")
    wid = sid * NC + cid

    pltpu.sync_copy(att_hbm, att_v)
    lane = lax.iota(jnp.int32, L)

    bufs = ((idx_src0, idx_dst0, xl_rows0, xr_rows0, msg_buf0, semA0, semA1, semW0),
            (idx_src1, idx_dst1, xl_rows1, xr_rows1, msg_buf1, semB0, semB1, semW1))

    for _, _, _, _, mb, _, _, _ in bufs:
        @pl.loop(0, CHUNK)
        def _(i):
            for j in range(W128 // L):
                mb[i, pl.ds(j * L, L)] = jnp.zeros((L,), jnp.float32)

    e_base0 = wid * MSG_TILE_E

    def issue(g, b):
        isrc, idst, xlr, xrr, _, s0, s1, _ = bufs[b]
        e_base = e_base0 + g * CHUNK
        pltpu.sync_copy(srcp_hbm.at[pl.ds(e_base, CHUNK)], isrc)
        pltpu.sync_copy(dstp_hbm.at[pl.ds(e_base, CHUNK)], idst)
        pltpu.async_copy(xl_hbm.at[isrc], xlr, s0)
        pltpu.async_copy(xr_hbm.at[idst], xrr, s1)

    def wait_gather(b):
        isrc, idst, xlr, xrr, _, s0, s1, _ = bufs[b]
        pltpu.make_async_copy(xl_hbm.at[isrc], xlr, s0).wait()
        pltpu.make_async_copy(xr_hbm.at[idst], xrr, s1).wait()

    def compute_store(g, b, first):
        _, _, xlr, xrr, mb, _, _, sw = bufs[b]
        e_base = e_base0 + g * CHUNK

        @pl.when(jnp.logical_not(first))
        def _():
            pltpu.make_async_copy(mb, msg_hbm.at[pl.ds(e_base, CHUNK)], sw).wait()

        @pl.loop(0, CHUNK)
        def _(i):
            lrow = jnp.full((L,), -60.0, jnp.float32)
            for h in range(HEADS):
                xj = xlr[i, pl.ds(h * L, L)]
                xi = xrr[i, pl.ds(h * L, L)]
                s = xi + xj
                e = jnp.maximum(s, s * 0.2)
                logit = jnp.sum(e * att_v[h])
                lv = jnp.broadcast_to(logit, (L,))
                mb[i, pl.ds(h * L, L)] = xj * jnp.exp(lv)
                lrow = jnp.where(lane == h, lv, lrow)
            mb[i, pl.ds(GAT_HID, L)] = jnp.exp(lrow)

        pltpu.async_copy(mb, msg_hbm.at[pl.ds(e_base, CHUNK)], sw)

    issue(0, 0)

    @pl.loop(0, MSG_CHUNKS // 2)
    def _(p):
        g = 2 * p
        issue(g + 1, 1)
        wait_gather(0)
        compute_store(g, 0, p == 0)

        @pl.when(g + 2 < MSG_CHUNKS)
        def _():
            issue(g + 2, 0)

        wait_gather(1)
        compute_store(g + 1, 1, p == 0)

    for b in range(2):
        _, _, _, _, mb, _, _, sw = bufs[b]
        g_last = MSG_CHUNKS - 2 + b
        e_base = e_base0 + g_last * CHUNK
        pltpu.make_async_copy(mb, msg_hbm.at[pl.ds(e_base, CHUNK)], sw).wait()


def _msg_phase(srcp, dstp, xl, xr, att):
    k = pl.kernel(
        _msg_body,
        out_type=jax.ShapeDtypeStruct((ET_PAD, W128), jnp.float32),
        mesh=_MESH,
        compiler_params=_SC_PARAMS,
        scratch_types=(
            2 * [pltpu.VMEM((CHUNK,), jnp.int32),
                 pltpu.VMEM((CHUNK,), jnp.int32),
                 pltpu.VMEM((CHUNK, W128), jnp.float32),
                 pltpu.VMEM((CHUNK, W128), jnp.float32),
                 pltpu.VMEM((CHUNK, W128), jnp.float32)]
            + [pltpu.VMEM((HEADS, L), jnp.float32)]
            + 6 * [pltpu.SemaphoreType.DMA]
        ),
    )
    return k(srcp, dstp, xl, xr, att)


def _scatter_body(dstp_hbm, msg_hbm, out_hbm,
                  idx_dst, idx_adj, rows, acc, sem0):
    cid = lax.axis_index("c")
    sid = lax.axis_index("s")
    tile_row0 = sid * Q_ROWS_PER_TILE
    e_base0 = sid * TILE_E

    for r in range(2):
        q = 2 * cid + r
        q_base = q * QP

        @pl.loop(0, CHUNK)
        def _(i):
            for j in range(W128 // L):
                rows[i, pl.ds(j * L, L)] = jnp.zeros((L,), jnp.float32)

        nfull = Q_ROWS_PER_TILE // CHUNK
        for k in range(nfull):
            pltpu.sync_copy(rows, acc.at[pl.ds(tile_row0 + k * CHUNK, CHUNK)])
        rem = Q_ROWS_PER_TILE - nfull * CHUNK
        if rem:
            pltpu.sync_copy(rows.at[pl.ds(0, rem)],
                            acc.at[pl.ds(tile_row0 + nfull * CHUNK, rem)])
        plsc.subcore_barrier()

        @pl.loop(0, N_CHUNK)
        def _(g):
            e_base = e_base0 + g * CHUNK
            pltpu.sync_copy(dstp_hbm.at[pl.ds(e_base, CHUNK)], idx_dst)
            cp0 = pltpu.async_copy(msg_hbm.at[pl.ds(e_base, CHUNK)], rows, sem0)

            @pl.loop(0, CHUNK // L)
            def _(j):
                d = idx_dst[pl.ds(j * L, L)]
                local = d - q_base
                ok = (local >= 0) & (local < QP)
                idx_adj[pl.ds(j * L, L)] = jnp.where(ok, local, DUMMY_Q)

            cp0.wait()
            pltpu.sync_copy(rows, acc.at[idx_adj], add=True)

        plsc.subcore_barrier()
        pltpu.sync_copy(acc.at[pl.ds(tile_row0, Q_ROWS_PER_TILE)],
                        out_hbm.at[pl.ds(q_base + tile_row0, Q_ROWS_PER_TILE)])
        plsc.subcore_barrier()


def _scatter_phase(dstp, msg):
    k = pl.kernel(
        _scatter_body,
        out_type=jax.ShapeDtypeStruct((NP, W128), jnp.float32),
        mesh=_MESH,
        compiler_params=_SC_PARAMS,
        scratch_types=[
            pltpu.VMEM((CHUNK,), jnp.int32),
            pltpu.VMEM((CHUNK,), jnp.int32),
            pltpu.VMEM((CHUNK, W128), jnp.float32),
            pltpu.VMEM_SHARED((QP, W128), jnp.float32),
            pltpu.SemaphoreType.DMA,
        ],
    )
    return k(dstp, msg)


def _edge_phase(srcp, dstp, xl, xr, att):
    msg = _msg_phase(srcp, dstp, xl, xr, att)
    return _scatter_phase(dstp, msg)


def _k1_body(lab_ref, typ_ref, scal_ref, labt_ref, typt_ref,
             wp_ref, bp_ref, wl_ref, bl_ref, wr_ref, br_ref,
             xl_ref, xr_ref):
    lab = lab_ref[...]
    typ = typ_ref[...]
    iota_l = lax.broadcasted_iota(jnp.int32, (BN, LAB_PAD), 1)
    iota_t = lax.broadcasted_iota(jnp.int32, (BN, TYP_PAD), 1)
    oh_l = (lab == iota_l).astype(jnp.float32)
    oh_t = (typ == iota_t).astype(jnp.float32)
    emb_l = jnp.dot(oh_l, labt_ref[...], preferred_element_type=jnp.float32)
    emb_t = jnp.dot(oh_t, typt_ref[...], preferred_element_type=jnp.float32)
    x = jnp.concatenate(
        [emb_l, emb_t, scal_ref[...],
         jnp.zeros((BN, DIN_PAD - 2 * EMB - 1), jnp.float32)], axis=1)
    xp = jnp.dot(x, wp_ref[...], preferred_element_type=jnp.float32) + bp_ref[...]
    xl_ref[...] = jnp.dot(xp, wl_ref[...],
                          preferred_element_type=jnp.float32) + bl_ref[...]
    xr_ref[...] = jnp.dot(xp, wr_ref[...],
                          preferred_element_type=jnp.float32) + br_ref[...]


def _k1(lab, typ, scal, labt, typt, wpT, bp, wlT, bl, wrT, br):
    full = lambda s: pl.BlockSpec(s, lambda i: (0, 0))
    return pl.pallas_call(
        _k1_body,
        grid=(NB,),
        in_specs=[
            pl.BlockSpec((BN, 1), lambda i: (i, 0)),
            pl.BlockSpec((BN, 1), lambda i: (i, 0)),
            pl.BlockSpec((BN, 1), lambda i: (i, 0)),
            full((LAB_PAD, EMB)),
            full((TYP_PAD, EMB)),
            full((DIN_PAD, GAT_HID)),
            full((1, GAT_HID)),
            full((GAT_HID, W128)),
            full((1, W128)),
            full((GAT_HID, W128)),
            full((1, W128)),
        ],
        out_specs=[
            pl.BlockSpec((BN, W128), lambda i: (i, 0)),
            pl.BlockSpec((BN, W128), lambda i: (i, 0)),
        ],
        out_shape=[
            jax.ShapeDtypeStruct((NP, W128), jnp.float32),
            jax.ShapeDtypeStruct((NP, W128), jnp.float32),
        ],
    )(lab, typ, scal, labt, typt, wpT, bp, wlT, bl, wrT, br)


def _gat_from_acc(acc, bias_row):
    numer = acc[:, :GAT_HID]
    den4 = acc[:, GAT_HID:GAT_HID + HEADS]
    dparts = [jnp.broadcast_to(den4[:, h][:, None], (acc.shape[0], L))
              for h in range(HEADS)]
    den = jnp.concatenate(dparts, axis=1)
    return numer / (den + 1e-16) + bias_row


def _valid_mask(i):
    r = i * BN + lax.broadcasted_iota(jnp.int32, (BN, 1), 0)
    ok = (r - (r // QP) * QP) < QR
    return ok.astype(jnp.float32)


def _sums_body(acc_ref, bias_ref, out_ref):
    i = pl.program_id(0)
    gat = _gat_from_acc(acc_ref[...], bias_ref[...])
    m = _valid_mask(i)
    g = gat * m
    s1 = jnp.sum(g, axis=0, keepdims=True)
    s2 = jnp.sum(g * gat, axis=0, keepdims=True)
    part = jnp.concatenate(
        [s1, s2, jnp.zeros((6, GAT_HID), jnp.float32)], axis=0)

    @pl.when(i == 0)
    def _():
        out_ref[...] = jnp.zeros_like(out_ref)

    out_ref[...] += part


def _sums(acc, bias_row):
    return pl.pallas_call(
        _sums_body,
        grid=(NB,),
        in_specs=[
            pl.BlockSpec((BN, W128), lambda i: (i, 0)),
            pl.BlockSpec((1, GAT_HID), lambda i: (0, 0)),
        ],
        out_specs=pl.BlockSpec((8, GAT_HID), lambda i: (0, 0)),
        out_shape=jax.ShapeDtypeStruct((8, GAT_HID), jnp.float32),
    )(acc, bias_row)


def _graph_norm_cols(gat, sums_ref, gw, gb, gms):
    s1 = sums_ref[0, :][None, :]
    s2 = sums_ref[1, :][None, :]
    mean = s1 / float(N)
    ex2 = s2 / float(N)
    var = ex2 - (2.0 * gms - gms * gms) * mean * mean
    out = gat - mean * gms
    return out * lax.rsqrt(var + 1e-5) * gw + gb


def _k2b_body(acc_ref, sums_ref, bias_ref, gw_ref, gb_ref, gms_ref,
              wl_ref, bl_ref, wr_ref, br_ref, wres_ref, bres_ref,
              xl_ref, xr_ref, res_ref):
    gat = _gat_from_acc(acc_ref[...], bias_ref[...])
    x1 = _graph_norm_cols(gat, sums_ref, gw_ref[...], gb_ref[...], gms_ref[...])
    x1 = jnp.where(x1 > 0, x1, jnp.exp(x1) - 1.0)
    xl_ref[...] = jnp.dot(x1, wl_ref[...],
                          preferred_element_type=jnp.float32) + bl_ref[...]
    xr_ref[...] = jnp.dot(x1, wr_ref[...],
                          preferred_element_type=jnp.float32) + br_ref[...]
    res_ref[...] = jnp.dot(x1, wres_ref[...],
                           preferred_element_type=jnp.float32) + bres_ref[...]


def _k2b(acc, sums, bias_row, gw, gb, gms, wlT, bl, wrT, br, wresT, bres):
    full = lambda s: pl.BlockSpec(s, lambda i: (0, 0))
    return pl.pallas_call(
        _k2b_body,
        grid=(NB,),
        in_specs=[
            pl.BlockSpec((BN, W128), lambda i: (i, 0)),
            full((8, GAT_HID)),
            full((1, GAT_HID)),
            full((1, GAT_HID)),
            full((1, GAT_HID)),
            full((1, GAT_HID)),
            full((GAT_HID, W128)),
            full((1, W128)),
            full((GAT_HID, W128)),
            full((1, W128)),
            full((GAT_HID, GAT_HID)),
            full((1, GAT_HID)),
        ],
        out_specs=[
            pl.BlockSpec((BN, W128), lambda i: (i, 0)),
            pl.BlockSpec((BN, W128), lambda i: (i, 0)),
            pl.BlockSpec((BN, GAT_HID), lambda i: (i, 0)),
        ],
        out_shape=[
            jax.ShapeDtypeStruct((NP, W128), jnp.float32),
            jax.ShapeDtypeStruct((NP, W128), jnp.float32),
            jax.ShapeDtypeStruct((NP, GAT_HID), jnp.float32),
        ],
    )(acc, sums, bias_row, gw, gb, gms, wlT, bl, wrT, br, wresT, bres)


def _k3b_body(acc_ref, sums_ref, bias_ref, gw_ref, gb_ref, gms_ref,
              res_ref, batch_ref, wih1_ref, b1_ref, wih2_ref, b2_ref,
              lnw_ref, lnb_ref, pooled_ref, counts_ref):
    i = pl.program_id(0)
    gat = _gat_from_acc(acc_ref[...], bias_ref[...])
    x2 = _graph_norm_cols(gat, sums_ref, gw_ref[...], gb_ref[...],
                          gms_ref[...]) + res_ref[...]
    x2 = jnp.where(x2 > 0, x2, jnp.exp(x2) - 1.0)

    def cell(x, wT, brow):
        g = jnp.dot(x, wT, preferred_element_type=jnp.float32) + brow
        gi = jax.nn.sigmoid(g[:, :LSTM_HID])
        gg = jnp.tanh(g[:, 2 * LSTM_HID:3 * LSTM_HID])
        go = jax.nn.sigmoid(g[:, 3 * LSTM_HID:])
        return go * jnp.tanh(gi * gg)

    h1 = cell(x2, wih1_ref[...], b1_ref[...])
    h2 = cell(h1, wih2_ref[...], b2_ref[...])

    mu = jnp.mean(h2, axis=1, keepdims=True)
    var = jnp.mean((h2 - mu) * (h2 - mu), axis=1, keepdims=True)
    xn = (h2 - mu) * lax.rsqrt(var + 1e-5) * lnw_ref[...] + lnb_ref[...]

    b = batch_ref[...]
    iota_g = lax.broadcasted_iota(jnp.int32, (BN, NUM_GRAPHS), 1)
    oh = (b == iota_g).astype(jnp.float32)
    pooled_part = lax.dot_general(oh, xn, (((0,), (0,)), ((), ())),
                                  preferred_element_type=jnp.float32)
    counts_part = lax.dot_general(oh, jnp.ones((BN, LSTM_HID), jnp.float32),
                                  (((0,), (0,)), ((), ())),
                                  preferred_element_type=jnp.float32)

    @pl.when(i == 0)
    def _():
        pooled_ref[...] = jnp.zeros_like(pooled_ref)
        counts_ref[...] = jnp.zeros_like(counts_ref)

    pooled_ref[...] += pooled_part
    counts_ref[...] += counts_part


def _k3b(acc, sums, bias_row, gw, gb, gms, res, batch_col,
         wih1T, b1, wih2T, b2, lnw, lnb):
    full = lambda s: pl.BlockSpec(s, lambda i: (0, 0))
    return pl.pallas_call(
        _k3b_body,
        grid=(NB,),
        in_specs=[
            pl.BlockSpec((BN, W128), lambda i: (i, 0)),
            full((8, GAT_HID)),
            full((1, GAT_HID)),
            full((1, GAT_HID)),
            full((1, GAT_HID)),
            full((1, GAT_HID)),
            pl.BlockSpec((BN, GAT_HID), lambda i: (i, 0)),
            pl.BlockSpec((BN, 1), lambda i: (i, 0)),
            full((GAT_HID, 4 * LSTM_HID)),
            full((1, 4 * LSTM_HID)),
            full((LSTM_HID, 4 * LSTM_HID)),
            full((1, 4 * LSTM_HID)),
            full((1, LSTM_HID)),
            full((1, LSTM_HID)),
        ],
        out_specs=[
            pl.BlockSpec((NUM_GRAPHS, LSTM_HID), lambda i: (0, 0)),
            pl.BlockSpec((NUM_GRAPHS, LSTM_HID), lambda i: (0, 0)),
        ],
        out_shape=[
            jax.ShapeDtypeStruct((NUM_GRAPHS, LSTM_HID), jnp.float32),
            jax.ShapeDtypeStruct((NUM_GRAPHS, LSTM_HID), jnp.float32),
        ],
    )(acc, sums, bias_row, gw, gb, gms, res, batch_col,
      wih1T, b1, wih2T, b2, lnw, lnb)


def _head_body(pooled_ref, counts_ref, wfc_ref, bfc_ref, out_ref):
    pooled = pooled_ref[...] / jnp.maximum(counts_ref[...], 1.0)
    logits = jnp.dot(pooled, wfc_ref[...],
                     preferred_element_type=jnp.float32) + bfc_ref[...]
    m = jnp.max(logits, axis=1, keepdims=True)
    s = jnp.log(jnp.sum(jnp.exp(logits - m), axis=1, keepdims=True))
    out_ref[...] = logits - m - s


def _head(pooled, counts, wfcT, bfc):
    return pl.pallas_call(
        _head_body,
        out_shape=jax.ShapeDtypeStruct((NUM_GRAPHS, OUT_DIM), jnp.float32),
    )(pooled, counts, wfcT, bfc)


def _pad_nodes(a, fill=0):
    pad = jnp.full((QP - QR,) + a.shape[1:], fill, a.dtype)
    parts = []
    for q in range(NQ):
        parts.append(a[q * QR:(q + 1) * QR])
        parts.append(pad)
    return jnp.concatenate(parts, axis=0)


def kernel(node_labels, node_types, node_scalar, edge_index, batch, label_table, type_table, Wp, bp, Wl1, bl1, Wr1, br1, att1, bias1, gn1_w, gn1_b, gn1_ms, Wl2, bl2, Wr2, br2, att2, bias2, gn2_w, gn2_b, gn2_ms, Wres, bres, Wih1, Whh1, bih1, bhh1, Wih2, Whh2, bih2, bhh2, ln_w, ln_b, Wfc, bfc):
    del Whh1, Whh2

    loops = jnp.arange(N, dtype=jnp.int32)
    src = jnp.concatenate([edge_index[0].astype(jnp.int32), loops])
    dst = jnp.concatenate([edge_index[1].astype(jnp.int32), loops])
    srcp = src + (QP - QR) * (src // QR)
    dstp = dst + (QP - QR) * (dst // QR)
    srcp = jnp.concatenate([srcp, jnp.zeros((ET_PAD - E_TOT,), jnp.int32)])
    dstp = jnp.concatenate([dstp, jnp.full((ET_PAD - E_TOT,), DUMMY_Q, jnp.int32)])

    lab_p = _pad_nodes(node_labels.astype(jnp.int32)[:, None])
    typ_p = _pad_nodes(node_types.astype(jnp.int32)[:, None])
    scal_p = _pad_nodes(node_scalar.astype(jnp.float32))
    batch_p = _pad_nodes(batch.astype(jnp.int32)[:, None], fill=-1)

    labt = jnp.zeros((LAB_PAD, EMB), jnp.float32).at[:NUM_LABELS].set(label_table)
    typt = jnp.zeros((TYP_PAD, EMB), jnp.float32).at[:NUM_TYPES].set(type_table)
    wpT = jnp.zeros((DIN_PAD, GAT_HID), jnp.float32).at[:2 * EMB + 1].set(Wp.T)
    row = lambda v: v[None, :]
    wide = lambda w: jnp.zeros((GAT_HID, W128), jnp.float32).at[:, :GAT_HID].set(w.T)
    widb = lambda b: jnp.zeros((1, W128), jnp.float32).at[:, :GAT_HID].set(b[None, :])

    xl1, xr1 = _k1(lab_p, typ_p, scal_p, labt, typt,
                   wpT, row(bp), wide(Wl1), widb(bl1), wide(Wr1), widb(br1))

    acc1 = _edge_phase(srcp, dstp, xl1, xr1, att1)
    sums1 = _sums(acc1, row(bias1))
    xl2, xr2, res = _k2b(acc1, sums1, row(bias1), row(gn1_w), row(gn1_b),
                         row(gn1_ms), wide(Wl2), widb(bl2), wide(Wr2),
                         widb(br2), Wres.T, row(bres))

    acc2 = _edge_phase(srcp, dstp, xl2, xr2, att2)
    sums2 = _sums(acc2, row(bias2))
    pooled, counts = _k3b(acc2, sums2, row(bias2), row(gn2_w), row(gn2_b),
                          row(gn2_ms), res, batch_p,
                          Wih1.T, row(bih1 + bhh1), Wih2.T, row(bih2 + bhh2),
                          row(ln_w), row(ln_b))

    return _head(pooled, counts, Wfc.T, row(bfc))


# speedup vs baseline: 26.5382x; 1.0967x over previous
import functools
import numpy as np
import jax
import jax.numpy as jnp
from jax import lax
from jax.experimental import pallas as pl
from jax.experimental.pallas import tpu as pltpu
from jax.experimental.pallas import tpu_sc as plsc

N = 50000
E = 800000
NUM_LABELS = 1000
NUM_TYPES = 100
EMB = 16
HEADS = 4
HEAD_DIM = 16
GAT_HID = HEADS * HEAD_DIM
LSTM_HID = 128
OUT_DIM = 4
NUM_GRAPHS = 64

NC = 2
NS = 16
L = 16
W128 = 128

QR = 12500
QP = 12544
NQ = 4
NP = NQ * QP
DUMMY_Q = 12520
Q_ROWS_PER_TILE = QP // NS

E_TOT = E + N
CHUNK = 128
TILE_E = 53248
N_CHUNK = TILE_E // CHUNK
SCH = 64
S_NCHUNK = TILE_E // SCH
ET_PAD = NS * TILE_E
MSG_TILE_E = ET_PAD // (NC * NS)
MSG_CHUNKS = MSG_TILE_E // CHUNK

NB = 16
BN = NP // NB

LAB_PAD = 1024
TYP_PAD = 128
DIN_PAD = 64


_MESH = plsc.VectorSubcoreMesh(core_axis_name="c", subcore_axis_name="s")
_SC_PARAMS = pltpu.CompilerParams(needs_layout_passes=False)


def _msg_body(srcp_hbm, dstp_hbm, xl_hbm, xr_hbm, att_hbm, msg_hbm,
              idx_src0, idx_dst0, xl_rows0, xr_rows0, msg_buf0,
              idx_src1, idx_dst1, xl_rows1, xr_rows1, msg_buf1, att_v,
              semA0, semA1, semB0, semB1, semW0, semW1):
    cid = lax.axis_index("c")
    sid = lax.axis_index("s")
    wid = sid * NC + cid

    pltpu.sync_copy(att_hbm, att_v)
    lane = lax.iota(jnp.int32, L)

    bufs = ((idx_src0, idx_dst0, xl_rows0, xr_rows0, msg_buf0, semA0, semA1, semW0),
            (idx_src1, idx_dst1, xl_rows1, xr_rows1, msg_buf1, semB0, semB1, semW1))

    for _, _, _, _, mb, _, _, _ in bufs:
        @pl.loop(0, CHUNK)
        def _(i):
            for j in range(W128 // L):
                mb[i, pl.ds(j * L, L)] = jnp.zeros((L,), jnp.float32)

    e_base0 = wid * MSG_TILE_E

    def issue(g, b):
        isrc, idst, xlr, xrr, _, s0, s1, _ = bufs[b]
        e_base = e_base0 + g * CHUNK
        pltpu.sync_copy(srcp_hbm.at[pl.ds(e_base, CHUNK)], isrc)
        pltpu.sync_copy(dstp_hbm.at[pl.ds(e_base, CHUNK)], idst)
        pltpu.async_copy(xl_hbm.at[isrc], xlr, s0)
        pltpu.async_copy(xr_hbm.at[idst], xrr, s1)

    def wait_gather(b):
        isrc, idst, xlr, xrr, _, s0, s1, _ = bufs[b]
        pltpu.make_async_copy(xl_hbm.at[isrc], xlr, s0).wait()
        pltpu.make_async_copy(xr_hbm.at[idst], xrr, s1).wait()

    def compute_store(g, b, first):
        _, _, xlr, xrr, mb, _, _, sw = bufs[b]
        e_base = e_base0 + g * CHUNK

        @pl.when(jnp.logical_not(first))
        def _():
            pltpu.make_async_copy(mb, msg_hbm.at[pl.ds(e_base, CHUNK)], sw).wait()

        @pl.loop(0, CHUNK)
        def _(i):
            lrow = jnp.full((L,), -60.0, jnp.float32)
            for h in range(HEADS):
                xj = xlr[i, pl.ds(h * L, L)]
                xi = xrr[i, pl.ds(h * L, L)]
                s = xi + xj
                e = jnp.maximum(s, s * 0.2)
                logit = jnp.sum(e * att_v[h])
                lv = jnp.broadcast_to(logit, (L,))
                mb[i, pl.ds(h * L, L)] = xj * jnp.exp(lv)
                lrow = jnp.where(lane == h, lv, lrow)
            mb[i, pl.ds(GAT_HID, L)] = jnp.exp(lrow)

        pltpu.async_copy(mb, msg_hbm.at[pl.ds(e_base, CHUNK)], sw)

    issue(0, 0)

    @pl.loop(0, MSG_CHUNKS // 2)
    def _(p):
        g = 2 * p
        issue(g + 1, 1)
        wait_gather(0)
        compute_store(g, 0, p == 0)

        @pl.when(g + 2 < MSG_CHUNKS)
        def _():
            issue(g + 2, 0)

        wait_gather(1)
        compute_store(g + 1, 1, p == 0)

    for b in range(2):
        _, _, _, _, mb, _, _, sw = bufs[b]
        g_last = MSG_CHUNKS - 2 + b
        e_base = e_base0 + g_last * CHUNK
        pltpu.make_async_copy(mb, msg_hbm.at[pl.ds(e_base, CHUNK)], sw).wait()


def _msg_phase(srcp, dstp, xl, xr, att):
    k = pl.kernel(
        _msg_body,
        out_type=jax.ShapeDtypeStruct((ET_PAD, W128), jnp.float32),
        mesh=_MESH,
        compiler_params=_SC_PARAMS,
        scratch_types=(
            2 * [pltpu.VMEM((CHUNK,), jnp.int32),
                 pltpu.VMEM((CHUNK,), jnp.int32),
                 pltpu.VMEM((CHUNK, W128), jnp.float32),
                 pltpu.VMEM((CHUNK, W128), jnp.float32),
                 pltpu.VMEM((CHUNK, W128), jnp.float32)]
            + [pltpu.VMEM((HEADS, L), jnp.float32)]
            + 6 * [pltpu.SemaphoreType.DMA]
        ),
    )
    return k(srcp, dstp, xl, xr, att)


def _scatter_body(dstp_hbm, msg_hbm, out_hbm,
                  idx_dst0, idx_adj0, rows0, idx_dst1, idx_adj1, rows1,
                  acc, semR0, semR1):
    cid = lax.axis_index("c")
    sid = lax.axis_index("s")
    tile_row0 = sid * Q_ROWS_PER_TILE
    e_base0 = sid * TILE_E

    bufs = ((idx_dst0, idx_adj0, rows0, semR0),
            (idx_dst1, idx_adj1, rows1, semR1))

    for r in range(2):
        q = 2 * cid + r
        q_base = q * QP

        @pl.loop(0, SCH)
        def _(i):
            for j in range(W128 // L):
                rows0[i, pl.ds(j * L, L)] = jnp.zeros((L,), jnp.float32)

        nfull = Q_ROWS_PER_TILE // SCH
        for k in range(nfull):
            pltpu.sync_copy(rows0, acc.at[pl.ds(tile_row0 + k * SCH, SCH)])
        rem = Q_ROWS_PER_TILE - nfull * SCH
        if rem:
            pltpu.sync_copy(rows0.at[pl.ds(0, rem)],
                            acc.at[pl.ds(tile_row0 + nfull * SCH, rem)])
        plsc.subcore_barrier()

        def issue(g, b):
            idst, iadj, rws, sr = bufs[b]
            e_base = e_base0 + g * SCH
            pltpu.sync_copy(dstp_hbm.at[pl.ds(e_base, SCH)], idst)
            pltpu.async_copy(msg_hbm.at[pl.ds(e_base, SCH)], rws, sr)

            @pl.loop(0, SCH // L)
            def _(j):
                d = idst[pl.ds(j * L, L)]
                local = d - q_base
                ok = (local >= 0) & (local < QP)
                iadj[pl.ds(j * L, L)] = jnp.where(ok, local, DUMMY_Q)

        def drain_scatter(b):
            idst, iadj, rws, sr = bufs[b]
            pltpu.make_async_copy(msg_hbm.at[pl.ds(0, SCH)], rws, sr).wait()
            pltpu.sync_copy(rws, acc.at[iadj], add=True)

        issue(0, 0)

        @pl.loop(0, S_NCHUNK // 2)
        def _(p):
            g = 2 * p
            issue(g + 1, 1)
            drain_scatter(0)

            @pl.when(g + 2 < S_NCHUNK)
            def _():
                issue(g + 2, 0)

            drain_scatter(1)

        plsc.subcore_barrier()
        pltpu.sync_copy(acc.at[pl.ds(tile_row0, Q_ROWS_PER_TILE)],
                        out_hbm.at[pl.ds(q_base + tile_row0, Q_ROWS_PER_TILE)])
        plsc.subcore_barrier()


def _scatter_phase(dstp, msg):
    k = pl.kernel(
        _scatter_body,
        out_type=jax.ShapeDtypeStruct((NP, W128), jnp.float32),
        mesh=_MESH,
        compiler_params=_SC_PARAMS,
        scratch_types=(
            2 * [pltpu.VMEM((SCH,), jnp.int32),
                 pltpu.VMEM((SCH,), jnp.int32),
                 pltpu.VMEM((SCH, W128), jnp.float32)]
            + [pltpu.VMEM_SHARED((QP, W128), jnp.float32)]
            + 2 * [pltpu.SemaphoreType.DMA]
        ),
    )
    return k(dstp, msg)


def _edge_phase(srcp, dstp, xl, xr, att):
    msg = _msg_phase(srcp, dstp, xl, xr, att)
    return _scatter_phase(dstp, msg)


def _k1_body(lab_ref, typ_ref, scal_ref, labt_ref, typt_ref,
             wp_ref, bp_ref, wl_ref, bl_ref, wr_ref, br_ref,
             xl_ref, xr_ref):
    lab = lab_ref[...]
    typ = typ_ref[...]
    iota_l = lax.broadcasted_iota(jnp.int32, (BN, LAB_PAD), 1)
    iota_t = lax.broadcasted_iota(jnp.int32, (BN, TYP_PAD), 1)
    oh_l = (lab == iota_l).astype(jnp.float32)
    oh_t = (typ == iota_t).astype(jnp.float32)
    emb_l = jnp.dot(oh_l, labt_ref[...], preferred_element_type=jnp.float32)
    emb_t = jnp.dot(oh_t, typt_ref[...], preferred_element_type=jnp.float32)
    x = jnp.concatenate(
        [emb_l, emb_t, scal_ref[...],
         jnp.zeros((BN, DIN_PAD - 2 * EMB - 1), jnp.float32)], axis=1)
    xp = jnp.dot(x, wp_ref[...], preferred_element_type=jnp.float32) + bp_ref[...]
    xl_ref[...] = jnp.dot(xp, wl_ref[...],
                          preferred_element_type=jnp.float32) + bl_ref[...]
    xr_ref[...] = jnp.dot(xp, wr_ref[...],
                          preferred_element_type=jnp.float32) + br_ref[...]


def _k1(lab, typ, scal, labt, typt, wpT, bp, wlT, bl, wrT, br):
    full = lambda s: pl.BlockSpec(s, lambda i: (0, 0))
    return pl.pallas_call(
        _k1_body,
        grid=(NB,),
        in_specs=[
            pl.BlockSpec((BN, 1), lambda i: (i, 0)),
            pl.BlockSpec((BN, 1), lambda i: (i, 0)),
            pl.BlockSpec((BN, 1), lambda i: (i, 0)),
            full((LAB_PAD, EMB)),
            full((TYP_PAD, EMB)),
            full((DIN_PAD, GAT_HID)),
            full((1, GAT_HID)),
            full((GAT_HID, W128)),
            full((1, W128)),
            full((GAT_HID, W128)),
            full((1, W128)),
        ],
        out_specs=[
            pl.BlockSpec((BN, W128), lambda i: (i, 0)),
            pl.BlockSpec((BN, W128), lambda i: (i, 0)),
        ],
        out_shape=[
            jax.ShapeDtypeStruct((NP, W128), jnp.float32),
            jax.ShapeDtypeStruct((NP, W128), jnp.float32),
        ],
    )(lab, typ, scal, labt, typt, wpT, bp, wlT, bl, wrT, br)


def _gat_from_acc(acc, bias_row):
    numer = acc[:, :GAT_HID]
    den4 = acc[:, GAT_HID:GAT_HID + HEADS]
    dparts = [jnp.broadcast_to(den4[:, h][:, None], (acc.shape[0], L))
              for h in range(HEADS)]
    den = jnp.concatenate(dparts, axis=1)
    return numer / (den + 1e-16) + bias_row


def _valid_mask(i):
    r = i * BN + lax.broadcasted_iota(jnp.int32, (BN, 1), 0)
    ok = (r - (r // QP) * QP) < QR
    return ok.astype(jnp.float32)


def _sums_body(acc_ref, bias_ref, out_ref):
    i = pl.program_id(0)
    gat = _gat_from_acc(acc_ref[...], bias_ref[...])
    m = _valid_mask(i)
    g = gat * m
    s1 = jnp.sum(g, axis=0, keepdims=True)
    s2 = jnp.sum(g * gat, axis=0, keepdims=True)
    part = jnp.concatenate(
        [s1, s2, jnp.zeros((6, GAT_HID), jnp.float32)], axis=0)

    @pl.when(i == 0)
    def _():
        out_ref[...] = jnp.zeros_like(out_ref)

    out_ref[...] += part


def _sums(acc, bias_row):
    return pl.pallas_call(
        _sums_body,
        grid=(NB,),
        in_specs=[
            pl.BlockSpec((BN, W128), lambda i: (i, 0)),
            pl.BlockSpec((1, GAT_HID), lambda i: (0, 0)),
        ],
        out_specs=pl.BlockSpec((8, GAT_HID), lambda i: (0, 0)),
        out_shape=jax.ShapeDtypeStruct((8, GAT_HID), jnp.float32),
    )(acc, bias_row)


def _graph_norm_cols(gat, sums_ref, gw, gb, gms):
    s1 = sums_ref[0, :][None, :]
    s2 = sums_ref[1, :][None, :]
    mean = s1 / float(N)
    ex2 = s2 / float(N)
    var = ex2 - (2.0 * gms - gms * gms) * mean * mean
    out = gat - mean * gms
    return out * lax.rsqrt(var + 1e-5) * gw + gb


def _k2b_body(acc_ref, sums_ref, bias_ref, gw_ref, gb_ref, gms_ref,
              wl_ref, bl_ref, wr_ref, br_ref, wres_ref, bres_ref,
              xl_ref, xr_ref, res_ref):
    gat = _gat_from_acc(acc_ref[...], bias_ref[...])
    x1 = _graph_norm_cols(gat, sums_ref, gw_ref[...], gb_ref[...], gms_ref[...])
    x1 = jnp.where(x1 > 0, x1, jnp.exp(x1) - 1.0)
    xl_ref[...] = jnp.dot(x1, wl_ref[...],
                          preferred_element_type=jnp.float32) + bl_ref[...]
    xr_ref[...] = jnp.dot(x1, wr_ref[...],
                          preferred_element_type=jnp.float32) + br_ref[...]
    res_ref[...] = jnp.dot(x1, wres_ref[...],
                           preferred_element_type=jnp.float32) + bres_ref[...]


def _k2b(acc, sums, bias_row, gw, gb, gms, wlT, bl, wrT, br, wresT, bres):
    full = lambda s: pl.BlockSpec(s, lambda i: (0, 0))
    return pl.pallas_call(
        _k2b_body,
        grid=(NB,),
        in_specs=[
            pl.BlockSpec((BN, W128), lambda i: (i, 0)),
            full((8, GAT_HID)),
            full((1, GAT_HID)),
            full((1, GAT_HID)),
            full((1, GAT_HID)),
            full((1, GAT_HID)),
            full((GAT_HID, W128)),
            full((1, W128)),
            full((GAT_HID, W128)),
            full((1, W128)),
            full((GAT_HID, GAT_HID)),
            full((1, GAT_HID)),
        ],
        out_specs=[
            pl.BlockSpec((BN, W128), lambda i: (i, 0)),
            pl.BlockSpec((BN, W128), lambda i: (i, 0)),
            pl.BlockSpec((BN, GAT_HID), lambda i: (i, 0)),
        ],
        out_shape=[
            jax.ShapeDtypeStruct((NP, W128), jnp.float32),
            jax.ShapeDtypeStruct((NP, W128), jnp.float32),
            jax.ShapeDtypeStruct((NP, GAT_HID), jnp.float32),
        ],
    )(acc, sums, bias_row, gw, gb, gms, wlT, bl, wrT, br, wresT, bres)


def _k3b_body(acc_ref, sums_ref, bias_ref, gw_ref, gb_ref, gms_ref,
              res_ref, batch_ref, wih1_ref, b1_ref, wih2_ref, b2_ref,
              lnw_ref, lnb_ref, pooled_ref, counts_ref):
    i = pl.program_id(0)
    gat = _gat_from_acc(acc_ref[...], bias_ref[...])
    x2 = _graph_norm_cols(gat, sums_ref, gw_ref[...], gb_ref[...],
                          gms_ref[...]) + res_ref[...]
    x2 = jnp.where(x2 > 0, x2, jnp.exp(x2) - 1.0)

    def cell(x, wT, brow):
        g = jnp.dot(x, wT, preferred_element_type=jnp.float32) + brow
        gi = jax.nn.sigmoid(g[:, :LSTM_HID])
        gg = jnp.tanh(g[:, 2 * LSTM_HID:3 * LSTM_HID])
        go = jax.nn.sigmoid(g[:, 3 * LSTM_HID:])
        return go * jnp.tanh(gi * gg)

    h1 = cell(x2, wih1_ref[...], b1_ref[...])
    h2 = cell(h1, wih2_ref[...], b2_ref[...])

    mu = jnp.mean(h2, axis=1, keepdims=True)
    var = jnp.mean((h2 - mu) * (h2 - mu), axis=1, keepdims=True)
    xn = (h2 - mu) * lax.rsqrt(var + 1e-5) * lnw_ref[...] + lnb_ref[...]

    b = batch_ref[...]
    iota_g = lax.broadcasted_iota(jnp.int32, (BN, NUM_GRAPHS), 1)
    oh = (b == iota_g).astype(jnp.float32)
    pooled_part = lax.dot_general(oh, xn, (((0,), (0,)), ((), ())),
                                  preferred_element_type=jnp.float32)
    counts_part = lax.dot_general(oh, jnp.ones((BN, LSTM_HID), jnp.float32),
                                  (((0,), (0,)), ((), ())),
                                  preferred_element_type=jnp.float32)

    @pl.when(i == 0)
    def _():
        pooled_ref[...] = jnp.zeros_like(pooled_ref)
        counts_ref[...] = jnp.zeros_like(counts_ref)

    pooled_ref[...] += pooled_part
    counts_ref[...] += counts_part


def _k3b(acc, sums, bias_row, gw, gb, gms, res, batch_col,
         wih1T, b1, wih2T, b2, lnw, lnb):
    full = lambda s: pl.BlockSpec(s, lambda i: (0, 0))
    return pl.pallas_call(
        _k3b_body,
        grid=(NB,),
        in_specs=[
            pl.BlockSpec((BN, W128), lambda i: (i, 0)),
            full((8, GAT_HID)),
            full((1, GAT_HID)),
            full((1, GAT_HID)),
            full((1, GAT_HID)),
            full((1, GAT_HID)),
            pl.BlockSpec((BN, GAT_HID), lambda i: (i, 0)),
            pl.BlockSpec((BN, 1), lambda i: (i, 0)),
            full((GAT_HID, 4 * LSTM_HID)),
            full((1, 4 * LSTM_HID)),
            full((LSTM_HID, 4 * LSTM_HID)),
            full((1, 4 * LSTM_HID)),
            full((1, LSTM_HID)),
            full((1, LSTM_HID)),
        ],
        out_specs=[
            pl.BlockSpec((NUM_GRAPHS, LSTM_HID), lambda i: (0, 0)),
            pl.BlockSpec((NUM_GRAPHS, LSTM_HID), lambda i: (0, 0)),
        ],
        out_shape=[
            jax.ShapeDtypeStruct((NUM_GRAPHS, LSTM_HID), jnp.float32),
            jax.ShapeDtypeStruct((NUM_GRAPHS, LSTM_HID), jnp.float32),
        ],
    )(acc, sums, bias_row, gw, gb, gms, res, batch_col,
      wih1T, b1, wih2T, b2, lnw, lnb)


def _head_body(pooled_ref, counts_ref, wfc_ref, bfc_ref, out_ref):
    pooled = pooled_ref[...] / jnp.maximum(counts_ref[...], 1.0)
    logits = jnp.dot(pooled, wfc_ref[...],
                     preferred_element_type=jnp.float32) + bfc_ref[...]
    m = jnp.max(logits, axis=1, keepdims=True)
    s = jnp.log(jnp.sum(jnp.exp(logits - m), axis=1, keepdims=True))
    out_ref[...] = logits - m - s


def _head(pooled, counts, wfcT, bfc):
    return pl.pallas_call(
        _head_body,
        out_shape=jax.ShapeDtypeStruct((NUM_GRAPHS, OUT_DIM), jnp.float32),
    )(pooled, counts, wfcT, bfc)


def _pad_nodes(a, fill=0):
    pad = jnp.full((QP - QR,) + a.shape[1:], fill, a.dtype)
    parts = []
    for q in range(NQ):
        parts.append(a[q * QR:(q + 1) * QR])
        parts.append(pad)
    return jnp.concatenate(parts, axis=0)


def kernel(node_labels, node_types, node_scalar, edge_index, batch, label_table, type_table, Wp, bp, Wl1, bl1, Wr1, br1, att1, bias1, gn1_w, gn1_b, gn1_ms, Wl2, bl2, Wr2, br2, att2, bias2, gn2_w, gn2_b, gn2_ms, Wres, bres, Wih1, Whh1, bih1, bhh1, Wih2, Whh2, bih2, bhh2, ln_w, ln_b, Wfc, bfc):
    del Whh1, Whh2

    loops = jnp.arange(N, dtype=jnp.int32)
    src = jnp.concatenate([edge_index[0].astype(jnp.int32), loops])
    dst = jnp.concatenate([edge_index[1].astype(jnp.int32), loops])
    srcp = src + (QP - QR) * (src // QR)
    dstp = dst + (QP - QR) * (dst // QR)
    srcp = jnp.concatenate([srcp, jnp.zeros((ET_PAD - E_TOT,), jnp.int32)])
    dstp = jnp.concatenate([dstp, jnp.full((ET_PAD - E_TOT,), DUMMY_Q, jnp.int32)])

    lab_p = _pad_nodes(node_labels.astype(jnp.int32)[:, None])
    typ_p = _pad_nodes(node_types.astype(jnp.int32)[:, None])
    scal_p = _pad_nodes(node_scalar.astype(jnp.float32))
    batch_p = _pad_nodes(batch.astype(jnp.int32)[:, None], fill=-1)

    labt = jnp.zeros((LAB_PAD, EMB), jnp.float32).at[:NUM_LABELS].set(label_table)
    typt = jnp.zeros((TYP_PAD, EMB), jnp.float32).at[:NUM_TYPES].set(type_table)
    wpT = jnp.zeros((DIN_PAD, GAT_HID), jnp.float32).at[:2 * EMB + 1].set(Wp.T)
    row = lambda v: v[None, :]
    wide = lambda w: jnp.zeros((GAT_HID, W128), jnp.float32).at[:, :GAT_HID].set(w.T)
    widb = lambda b: jnp.zeros((1, W128), jnp.float32).at[:, :GAT_HID].set(b[None, :])

    xl1, xr1 = _k1(lab_p, typ_p, scal_p, labt, typt,
                   wpT, row(bp), wide(Wl1), widb(bl1), wide(Wr1), widb(br1))

    acc1 = _edge_phase(srcp, dstp, xl1, xr1, att1)
    sums1 = _sums(acc1, row(bias1))
    xl2, xr2, res = _k2b(acc1, sums1, row(bias1), row(gn1_w), row(gn1_b),
                         row(gn1_ms), wide(Wl2), widb(bl2), wide(Wr2),
                         widb(br2), Wres.T, row(bres))

    acc2 = _edge_phase(srcp, dstp, xl2, xr2, att2)
    sums2 = _sums(acc2, row(bias2))
    pooled, counts = _k3b(acc2, sums2, row(bias2), row(gn2_w), row(gn2_b),
                          row(gn2_ms), res, batch_p,
                          Wih1.T, row(bih1 + bhh1), Wih2.T, row(bih2 + bhh2),
                          row(ln_w), row(ln_b))

    return _head(pooled, counts, Wfc.T, row(bfc))
